# baseline (speedup 1.0000x reference)
import jax
import jax.numpy as jnp
from jax.experimental import pallas as pl

B, S, D = 4, 4096, 1024
BLOCK_S = 2048
GRID = (B, S // BLOCK_S)
LANES = 128
DG = D // LANES


def _select_body(mask_ref, emb_ref, x_ref, out_ref):
    m = mask_ref[0]
    x3 = x_ref[0].reshape(BLOCK_S, DG, LANES)
    emb3 = emb_ref[...].reshape(1, DG, LANES)
    out = jnp.where(m[:, None, :], emb3, x3)
    out_ref[...] = out.reshape(1, BLOCK_S, D)


def kernel(x, mask_embedding):
    mask = jax.random.normal(jax.random.key(0), (B, S), dtype=jnp.float32) > 0.5
    m128 = jnp.broadcast_to(mask[..., None], (B, S, LANES))
    emb = mask_embedding.astype(x.dtype).reshape(1, D)

    out = pl.pallas_call(
        _select_body,
        grid=GRID,
        in_specs=[
            pl.BlockSpec((1, BLOCK_S, LANES), lambda b, s: (b, s, 0)),
            pl.BlockSpec((1, D), lambda b, s: (0, 0)),
            pl.BlockSpec((1, BLOCK_S, D), lambda b, s: (b, s, 0)),
        ],
        out_specs=pl.BlockSpec((1, BLOCK_S, D), lambda b, s: (b, s, 0)),
        out_shape=jax.ShapeDtypeStruct((B, S, D), x.dtype),
    )(m128, emb, x)

    return out, mask


# speedup vs baseline: 1.2879x; 1.2879x over previous
import jax
import jax.numpy as jnp
from jax.experimental import pallas as pl

B, S, D = 4, 4096, 1024
BLOCK_S = 2048
GRID = (B, S // BLOCK_S)


def _select_body(mask_ref, emb_ref, x_ref, out_ref):
    m = mask_ref[0, 0]
    mt = jnp.transpose(m, (1, 0))
    out_ref[...] = jnp.where(mt != 0.0, emb_ref[...], x_ref[0])[None]


def kernel(x, mask_embedding):
    mask = jax.random.normal(jax.random.key(0), (B, S), dtype=jnp.float32) > 0.5
    m4 = mask.astype(jnp.float32).reshape(B, S // BLOCK_S, 1, BLOCK_S)
    emb = mask_embedding.astype(x.dtype).reshape(1, D)

    out = pl.pallas_call(
        _select_body,
        grid=GRID,
        in_specs=[
            pl.BlockSpec((1, 1, 1, BLOCK_S), lambda b, s: (b, s, 0, 0)),
            pl.BlockSpec((1, D), lambda b, s: (0, 0)),
            pl.BlockSpec((1, BLOCK_S, D), lambda b, s: (b, s, 0)),
        ],
        out_specs=pl.BlockSpec((1, BLOCK_S, D), lambda b, s: (b, s, 0)),
        out_shape=jax.ShapeDtypeStruct((B, S, D), x.dtype),
    )(m4, emb, x)

    return out, mask


# speedup vs baseline: 1.3829x; 1.0738x over previous
import jax
import jax.numpy as jnp
from jax.experimental import pallas as pl

B, S, D = 4, 4096, 1024
BLOCK_S = 512
GRID = (S // BLOCK_S,)


def _select_body(mask_ref, emb_ref, x_ref, out_ref):
    m = mask_ref[0]
    mt = jnp.transpose(m, (1, 0))[None]
    cond = jnp.transpose(mt, (2, 1, 0))
    out_ref[...] = jnp.where(cond != 0.0, emb_ref[...], x_ref[...])


def kernel(x, mask_embedding):
    mask = jax.random.normal(jax.random.key(0), (B, S), dtype=jnp.float32) > 0.5
    m3 = mask.astype(jnp.float32)[None]
    emb = mask_embedding.astype(x.dtype).reshape(1, 1, D)

    out = pl.pallas_call(
        _select_body,
        grid=GRID,
        in_specs=[
            pl.BlockSpec((1, B, BLOCK_S), lambda s: (0, 0, s)),
            pl.BlockSpec((1, 1, D), lambda s: (0, 0, 0)),
            pl.BlockSpec((B, BLOCK_S, D), lambda s: (0, s, 0)),
        ],
        out_specs=pl.BlockSpec((B, BLOCK_S, D), lambda s: (0, s, 0)),
        out_shape=jax.ShapeDtypeStruct((B, S, D), x.dtype),
    )(m3, emb, x)

    return out, mask
